# baseline (speedup 1.0000x reference)
import functools

import jax
import jax.numpy as jnp
from jax import lax
from jax.experimental import pallas as pl
from jax.experimental.pallas import tpu as pltpu
from jax.experimental.pallas import tpu_sc as plsc

N = 10000
M = 10000
E = 160000

NC = 2
NS = 16
NW = NC * NS
CHUNK = 128
EPAD = 163840
NCH = EPAD // (NW * CHUNK)
NPAD = 10112
RPT = NPAD // NS


def _sc_segsum(table, gidx, sidx, zeros, D):
    mesh = plsc.VectorSubcoreMesh(core_axis_name="c", subcore_axis_name="s",
                                  num_cores=NC, num_subcores=NS)

    @functools.partial(
        pl.kernel,
        out_type=jax.ShapeDtypeStruct((NC, N, D), jnp.float32),
        mesh=mesh,
        compiler_params=pltpu.CompilerParams(
            use_tc_tiling_on_sc=False,
            disable_bounds_checks=True,
            disable_semaphore_checks=True,
        ),
        scratch_types=[
            pltpu.VMEM((NCH, CHUNK), jnp.int32),
            pltpu.VMEM((NCH, CHUNK), jnp.int32),
            pltpu.VMEM((NCH, CHUNK, D), jnp.float32),
            pltpu.VMEM_SHARED((NPAD, D), jnp.float32),
            pltpu.VMEM_SHARED((N, D), jnp.float32),
            pltpu.SemaphoreType.DMA,
            pltpu.SemaphoreType.DMA,
        ],
    )
    def k(table_h, gidx_h, sidx_h, zeros_h, out_h, gv, sv, rows, accum,
          shtab, sem, sem2):
        c = lax.axis_index("c")
        s = lax.axis_index("s")
        wid = s * NC + c
        pltpu.sync_copy(table_h.at[pl.ds(s * 625, 625)],
                        shtab.at[pl.ds(s * 625, 625)])
        pltpu.sync_copy(gidx_h.at[wid], gv)
        pltpu.sync_copy(sidx_h.at[wid], sv)
        pltpu.sync_copy(zeros_h.at[pl.ds(s * RPT, RPT)],
                        accum.at[pl.ds(s * RPT, RPT)])

        plsc.subcore_barrier()

        def fire(j, carry):
            pltpu.async_copy(shtab.at[gv.at[j]], rows.at[j], sem)
            return carry
        lax.fori_loop(0, NCH, fire, 0)

        def drain(j, carry):
            pltpu.make_async_copy(shtab.at[gv.at[j]], rows.at[j], sem).wait()
            return carry
        lax.fori_loop(0, NCH, drain, 0)

        def scat(j, carry):
            pltpu.async_copy(rows.at[j], accum.at[sv.at[j]], sem2, add=True)
            return carry
        lax.fori_loop(0, NCH, scat, 0)

        def sdrain(j, carry):
            pltpu.make_async_copy(rows.at[j], accum.at[sv.at[j]],
                                  sem2).wait()
            return carry
        lax.fori_loop(0, NCH, sdrain, 0)

        plsc.subcore_barrier()
        @pl.when(s < 10)
        def _():
            pltpu.sync_copy(accum.at[pl.ds(s * 1000, 1000)],
                            out_h.at[c].at[pl.ds(s * 1000, 1000)])

    return k(table, gidx, sidx, zeros)


_RA = 5000
_GA = (N // _RA,)
_R = 10000
_G5 = (N // _R,)


def _full(shape):
    return pl.BlockSpec(shape, lambda i: tuple(0 for _ in shape))


def _rows(shape):
    if len(shape) == 2:
        return pl.BlockSpec(shape, lambda i: (i, 0))
    return pl.BlockSpec(shape, lambda i: (0, i, 0))


def _tc_a(x, w, Wx, bx, Ww1, bw1, Ww2, bw2):
    def body(x_r, w_r, Wx_r, bx_r, Ww1_r, bw1_r, Ww2_r, bw2_r,
             xa_r, wa_r, wb_r):
        xa_r[...] = jnp.dot(x_r[...], Wx_r[...],
                            preferred_element_type=jnp.float32) + bx_r[...]
        wa_r[...] = jnp.dot(w_r[...], Ww1_r[...],
                            preferred_element_type=jnp.float32) + bw1_r[...]
        wb_r[...] = jnp.dot(w_r[...], Ww2_r[...],
                            preferred_element_type=jnp.float32) + bw2_r[...]

    return pl.pallas_call(
        body,
        grid=_GA,
        in_specs=[_rows((_RA, 256)), _rows((_RA, 128)),
                  _full((256, 8)), _full((1, 8)),
                  _full((128, 8)), _full((1, 8)),
                  _full((128, 8)), _full((1, 8))],
        out_specs=[_rows((_RA, 8)), _rows((_RA, 8)), _rows((_RA, 8))],
        out_shape=[jax.ShapeDtypeStruct((N, 8), jnp.float32)] * 3,
    )(x, w, Wx, bx, Ww1, bw1, Ww2, bw2)


def _tc_b(xa, parts, W2, b2):
    def body(xa_r, p_r, W2_r, b2_r, x1_r, xc_r):
        x1 = xa_r[...] * (1.0 + p_r[0] + p_r[1])
        x1_r[...] = x1
        xc_r[...] = jnp.dot(x1, W2_r[...],
                            preferred_element_type=jnp.float32) + b2_r[...]

    return pl.pallas_call(
        body,
        grid=_G5,
        in_specs=[_rows((_R, 8)), _rows((NC, _R, 8)),
                  _full((8, 8)), _full((1, 8))],
        out_specs=[_rows((_R, 8)), _rows((_R, 8))],
        out_shape=[jax.ShapeDtypeStruct((N, 8), jnp.float32)] * 2,
    )(xa, parts, W2, b2)


def _tc_c(x1, wb, parts, Wd, bd, We, be, Wxe, bxe):
    def body(x1_r, wb_r, p_r, Wd_r, bd_r, We_r, be_r, Wxe_r, bxe_r,
             wd_r, we_r, xe_r):
        w1 = wb_r[...] * (1.0 + p_r[0] + p_r[1])
        w1s = jax.nn.sigmoid(w1)
        wd_r[...] = jnp.dot(w1s, Wd_r[...],
                            preferred_element_type=jnp.float32) + bd_r[...]
        we_r[...] = jnp.dot(w1s, We_r[...],
                            preferred_element_type=jnp.float32) + be_r[...]
        x1s = jax.nn.sigmoid(x1_r[...])
        xe_r[...] = jnp.dot(x1s, Wxe_r[...],
                            preferred_element_type=jnp.float32) + bxe_r[...]

    return pl.pallas_call(
        body,
        grid=_G5,
        in_specs=[_rows((_R, 8)), _rows((_R, 8)), _rows((NC, _R, 8)),
                  _full((8, 8)), _full((1, 8)),
                  _full((8, 4)), _full((1, 4)),
                  _full((8, 4)), _full((1, 4))],
        out_specs=[_rows((_R, 8)), _rows((_R, 4)), _rows((_R, 4))],
        out_shape=[jax.ShapeDtypeStruct((M, 8), jnp.float32),
                   jax.ShapeDtypeStruct((M, 4), jnp.float32),
                   jax.ShapeDtypeStruct((M, 4), jnp.float32)],
    )(x1, wb, parts, Wd, bd, We, be, Wxe, bxe)


def _tc_d(xe, parts, Wf, bf):
    def body(xe_r, p_r, Wf_r, bf_r, xf_r, x2s_r):
        x2 = xe_r[...] * (1.0 + p_r[0, :, :4] + p_r[1, :, :4])
        xf_r[...] = jnp.dot(x2, Wf_r[...],
                            preferred_element_type=jnp.float32) + bf_r[...]
        x2s_r[...] = jax.nn.sigmoid(x2)

    return pl.pallas_call(
        body,
        grid=_G5,
        in_specs=[_rows((_R, 4)), _rows((NC, _R, 8)),
                  _full((4, 8)), _full((1, 8))],
        out_specs=[_rows((_R, 8)), _rows((_R, 4))],
        out_shape=[jax.ShapeDtypeStruct((N, 8), jnp.float32),
                   jax.ShapeDtypeStruct((N, 4), jnp.float32)],
    )(xe, parts, Wf, bf)


def _tc_e(we, parts):
    def body(we_r, p_r, w2s_r):
        w2s_r[...] = jax.nn.sigmoid(
            we_r[...] * (1.0 + p_r[0, :, :4] + p_r[1, :, :4]))

    return pl.pallas_call(
        body,
        grid=_G5,
        in_specs=[_rows((_R, 4)), _rows((NC, _R, 8))],
        out_specs=_rows((_R, 4)),
        out_shape=jax.ShapeDtypeStruct((M, 4), jnp.float32),
    )(we, parts)


def kernel(x, h, w,
           c1_mtx_Wx, c1_mtx_bx, c1_mtx_Ww, c1_mtx_bw,
           c1_mte_Wx, c1_mte_bx, c1_mte_Ww, c1_mte_bw,
           c2_mtx_Wx, c2_mtx_bx, c2_mtx_Ww, c2_mtx_bw,
           c2_mte_Wx, c2_mte_bx, c2_mte_Ww, c2_mte_bw):
    h = h.astype(jnp.int32)
    pad = EPAD - E
    g0 = jnp.pad(h[0], (0, pad)).reshape(NW, NCH, CHUNK)
    s0 = jnp.pad(h[0], (0, pad), constant_values=N).reshape(NW, NCH, CHUNK)
    g1 = jnp.pad(h[1], (0, pad)).reshape(NW, NCH, CHUNK)
    s1 = jnp.pad(h[1], (0, pad), constant_values=N).reshape(NW, NCH, CHUNK)
    z8 = jnp.zeros((NPAD, 8), jnp.float32)

    r = lambda b: b.reshape(1, -1)
    p8 = lambda W: jnp.pad(W, ((0, 0), (0, 8 - W.shape[1])))

    xa, wa, wb = _tc_a(x, w, c1_mtx_Wx, r(c1_mtx_bx), c1_mtx_Ww, r(c1_mtx_bw),
                       c1_mte_Ww, r(c1_mte_bw))
    p1 = _sc_segsum(wa, g1, s0, z8, 8)
    x1, xc = _tc_b(xa, p1, c1_mte_Wx, r(c1_mte_bx))
    p2 = _sc_segsum(xc, g0, s1, z8, 8)
    wd, we, xe = _tc_c(x1, wb, p2, p8(c2_mtx_Ww), r(p8(c2_mtx_bw[None])),
                       c2_mte_Ww, r(c2_mte_bw), c2_mtx_Wx, r(c2_mtx_bx))
    p3 = _sc_segsum(wd, g1, s0, z8, 8)
    xf, x2s = _tc_d(xe, p3, p8(c2_mte_Wx), r(p8(c2_mte_bx[None])))
    p4 = _sc_segsum(xf, g0, s1, z8, 8)
    w2s = _tc_e(we, p4)
    return (x2s, w2s)


# speedup vs baseline: 1.0014x; 1.0014x over previous
import functools

import jax
import jax.numpy as jnp
from jax import lax
from jax.experimental import pallas as pl
from jax.experimental.pallas import tpu as pltpu
from jax.experimental.pallas import tpu_sc as plsc

N = 10000
M = 10000
E = 160000

NC = 2
NS = 16
NW = NC * NS
CHUNK = 128
EPAD = 163840
NCH = EPAD // (NW * CHUNK)
NPAD = 10112
RPT = NPAD // NS


def _sc_segsum(table, gidx, sidx, zeros, D):
    mesh = plsc.VectorSubcoreMesh(core_axis_name="c", subcore_axis_name="s",
                                  num_cores=NC, num_subcores=NS)

    @functools.partial(
        pl.kernel,
        out_type=jax.ShapeDtypeStruct((NC, N, D), jnp.float32),
        mesh=mesh,
        compiler_params=pltpu.CompilerParams(use_tc_tiling_on_sc=False),
        scratch_types=[
            pltpu.VMEM((NCH, CHUNK), jnp.int32),
            pltpu.VMEM((NCH, CHUNK), jnp.int32),
            pltpu.VMEM((NCH, CHUNK, D), jnp.float32),
            pltpu.VMEM_SHARED((NPAD, D), jnp.float32),
            pltpu.VMEM_SHARED((N, D), jnp.float32),
            pltpu.SemaphoreType.DMA,
            pltpu.SemaphoreType.DMA,
        ],
    )
    def k(table_h, gidx_h, sidx_h, zeros_h, out_h, gv, sv, rows, accum,
          shtab, sem, sem2):
        c = lax.axis_index("c")
        s = lax.axis_index("s")
        wid = s * NC + c
        pltpu.sync_copy(table_h.at[pl.ds(s * 625, 625)],
                        shtab.at[pl.ds(s * 625, 625)])
        pltpu.sync_copy(gidx_h.at[wid], gv)
        pltpu.sync_copy(sidx_h.at[wid], sv)
        pltpu.sync_copy(zeros_h.at[pl.ds(s * RPT, RPT)],
                        accum.at[pl.ds(s * RPT, RPT)])

        plsc.subcore_barrier()

        def fire(j, carry):
            pltpu.async_copy(shtab.at[gv.at[j]], rows.at[j], sem)
            return carry
        lax.fori_loop(0, NCH, fire, 0)

        def drain(j, carry):
            pltpu.make_async_copy(shtab.at[gv.at[j]], rows.at[j], sem).wait()
            return carry
        lax.fori_loop(0, NCH, drain, 0)

        def scat(j, carry):
            pltpu.async_copy(rows.at[j], accum.at[sv.at[j]], sem2, add=True)
            return carry
        lax.fori_loop(0, NCH, scat, 0)

        def sdrain(j, carry):
            pltpu.make_async_copy(rows.at[j], accum.at[sv.at[j]],
                                  sem2).wait()
            return carry
        lax.fori_loop(0, NCH, sdrain, 0)

        plsc.subcore_barrier()
        @pl.when(s < 10)
        def _():
            pltpu.sync_copy(accum.at[pl.ds(s * 1000, 1000)],
                            out_h.at[c].at[pl.ds(s * 1000, 1000)])

    return k(table, gidx, sidx, zeros)


_RA = 5000
_GA = (N // _RA,)
_R = 10000
_G5 = (N // _R,)


def _full(shape):
    return pl.BlockSpec(shape, lambda i: tuple(0 for _ in shape))


def _rows(shape):
    if len(shape) == 2:
        return pl.BlockSpec(shape, lambda i: (i, 0))
    return pl.BlockSpec(shape, lambda i: (0, i, 0))


def _tc_a(x, w, Wx, bx, Ww1, bw1, Ww2, bw2):
    def body(x_r, w_r, Wx_r, bx_r, Ww1_r, bw1_r, Ww2_r, bw2_r,
             xa_r, wa_r, wb_r):
        xa_r[...] = jnp.dot(x_r[...], Wx_r[...],
                            preferred_element_type=jnp.float32) + bx_r[...]
        wa_r[...] = jnp.dot(w_r[...], Ww1_r[...],
                            preferred_element_type=jnp.float32) + bw1_r[...]
        wb_r[...] = jnp.dot(w_r[...], Ww2_r[...],
                            preferred_element_type=jnp.float32) + bw2_r[...]

    return pl.pallas_call(
        body,
        grid=_GA,
        in_specs=[_rows((_RA, 256)), _rows((_RA, 128)),
                  _full((256, 8)), _full((1, 8)),
                  _full((128, 8)), _full((1, 8)),
                  _full((128, 8)), _full((1, 8))],
        out_specs=[_rows((_RA, 8)), _rows((_RA, 8)), _rows((_RA, 8))],
        out_shape=[jax.ShapeDtypeStruct((N, 8), jnp.float32)] * 3,
    )(x, w, Wx, bx, Ww1, bw1, Ww2, bw2)


def _tc_b(xa, parts, W2, b2):
    def body(xa_r, p_r, W2_r, b2_r, x1_r, xc_r):
        x1 = xa_r[...] * (1.0 + p_r[0] + p_r[1])
        x1_r[...] = x1
        xc_r[...] = jnp.dot(x1, W2_r[...],
                            preferred_element_type=jnp.float32) + b2_r[...]

    return pl.pallas_call(
        body,
        grid=_G5,
        in_specs=[_rows((_R, 8)), _rows((NC, _R, 8)),
                  _full((8, 8)), _full((1, 8))],
        out_specs=[_rows((_R, 8)), _rows((_R, 8))],
        out_shape=[jax.ShapeDtypeStruct((N, 8), jnp.float32)] * 2,
    )(xa, parts, W2, b2)


def _tc_c(x1, wb, parts, Wd, bd, We, be, Wxe, bxe):
    def body(x1_r, wb_r, p_r, Wd_r, bd_r, We_r, be_r, Wxe_r, bxe_r,
             wd_r, we_r, xe_r):
        w1 = wb_r[...] * (1.0 + p_r[0] + p_r[1])
        w1s = jax.nn.sigmoid(w1)
        wd_r[...] = jnp.dot(w1s, Wd_r[...],
                            preferred_element_type=jnp.float32) + bd_r[...]
        we_r[...] = jnp.dot(w1s, We_r[...],
                            preferred_element_type=jnp.float32) + be_r[...]
        x1s = jax.nn.sigmoid(x1_r[...])
        xe_r[...] = jnp.dot(x1s, Wxe_r[...],
                            preferred_element_type=jnp.float32) + bxe_r[...]

    return pl.pallas_call(
        body,
        grid=_G5,
        in_specs=[_rows((_R, 8)), _rows((_R, 8)), _rows((NC, _R, 8)),
                  _full((8, 8)), _full((1, 8)),
                  _full((8, 4)), _full((1, 4)),
                  _full((8, 4)), _full((1, 4))],
        out_specs=[_rows((_R, 8)), _rows((_R, 4)), _rows((_R, 4))],
        out_shape=[jax.ShapeDtypeStruct((M, 8), jnp.float32),
                   jax.ShapeDtypeStruct((M, 4), jnp.float32),
                   jax.ShapeDtypeStruct((M, 4), jnp.float32)],
    )(x1, wb, parts, Wd, bd, We, be, Wxe, bxe)


def _tc_d(xe, parts, Wf, bf):
    def body(xe_r, p_r, Wf_r, bf_r, xf_r, x2s_r):
        x2 = xe_r[...] * (1.0 + p_r[0, :, :4] + p_r[1, :, :4])
        xf_r[...] = jnp.dot(x2, Wf_r[...],
                            preferred_element_type=jnp.float32) + bf_r[...]
        x2s_r[...] = jax.nn.sigmoid(x2)

    return pl.pallas_call(
        body,
        grid=_G5,
        in_specs=[_rows((_R, 4)), _rows((NC, _R, 8)),
                  _full((4, 8)), _full((1, 8))],
        out_specs=[_rows((_R, 8)), _rows((_R, 4))],
        out_shape=[jax.ShapeDtypeStruct((N, 8), jnp.float32),
                   jax.ShapeDtypeStruct((N, 4), jnp.float32)],
    )(xe, parts, Wf, bf)


def _tc_e(we, parts):
    def body(we_r, p_r, w2s_r):
        w2s_r[...] = jax.nn.sigmoid(
            we_r[...] * (1.0 + p_r[0, :, :4] + p_r[1, :, :4]))

    return pl.pallas_call(
        body,
        grid=_G5,
        in_specs=[_rows((_R, 4)), _rows((NC, _R, 8))],
        out_specs=_rows((_R, 4)),
        out_shape=jax.ShapeDtypeStruct((M, 4), jnp.float32),
    )(we, parts)


def kernel(x, h, w,
           c1_mtx_Wx, c1_mtx_bx, c1_mtx_Ww, c1_mtx_bw,
           c1_mte_Wx, c1_mte_bx, c1_mte_Ww, c1_mte_bw,
           c2_mtx_Wx, c2_mtx_bx, c2_mtx_Ww, c2_mtx_bw,
           c2_mte_Wx, c2_mte_bx, c2_mte_Ww, c2_mte_bw):
    h = h.astype(jnp.int32)
    pad = EPAD - E
    g0 = jnp.pad(h[0], (0, pad)).reshape(NW, NCH, CHUNK)
    s0 = jnp.pad(h[0], (0, pad), constant_values=N).reshape(NW, NCH, CHUNK)
    g1 = jnp.pad(h[1], (0, pad)).reshape(NW, NCH, CHUNK)
    s1 = jnp.pad(h[1], (0, pad), constant_values=N).reshape(NW, NCH, CHUNK)
    z8 = jnp.zeros((NPAD, 8), jnp.float32)

    r = lambda b: b.reshape(1, -1)
    p8 = lambda W: jnp.pad(W, ((0, 0), (0, 8 - W.shape[1])))

    xa, wa, wb = _tc_a(x, w, c1_mtx_Wx, r(c1_mtx_bx), c1_mtx_Ww, r(c1_mtx_bw),
                       c1_mte_Ww, r(c1_mte_bw))
    p1 = _sc_segsum(wa, g1, s0, z8, 8)
    x1, xc = _tc_b(xa, p1, c1_mte_Wx, r(c1_mte_bx))
    p2 = _sc_segsum(xc, g0, s1, z8, 8)
    wd, we, xe = _tc_c(x1, wb, p2, p8(c2_mtx_Ww), r(p8(c2_mtx_bw[None])),
                       c2_mte_Ww, r(c2_mte_bw), c2_mtx_Wx, r(c2_mtx_bx))
    p3 = _sc_segsum(wd, g1, s0, z8, 8)
    xf, x2s = _tc_d(xe, p3, p8(c2_mte_Wx), r(p8(c2_mte_bx[None])))
    p4 = _sc_segsum(xf, g0, s1, z8, 8)
    w2s = _tc_e(we, p4)
    return (x2s, w2s)


# speedup vs baseline: 1.5661x; 1.5638x over previous
import functools

import jax
import jax.numpy as jnp
from jax import lax
from jax.experimental import pallas as pl
from jax.experimental.pallas import tpu as pltpu
from jax.experimental.pallas import tpu_sc as plsc

N = 10000
M = 10000
E = 160000

NC = 2
NS = 16
NW = NC * NS
CHUNK = 128
EPAD = 163840
NCH = EPAD // (NW * CHUNK)
NPAD = 10112
RPT = NPAD // NS


def _sc_segsum(table, gidx, sidx, zeros, D):
    mesh = plsc.VectorSubcoreMesh(core_axis_name="c", subcore_axis_name="s",
                                  num_cores=NC, num_subcores=NS)

    @functools.partial(
        pl.kernel,
        out_type=jax.ShapeDtypeStruct((NC, N, D), jnp.float32),
        mesh=mesh,
        compiler_params=pltpu.CompilerParams(use_tc_tiling_on_sc=False),
        scratch_types=[
            pltpu.VMEM((NCH, CHUNK), jnp.int32),
            pltpu.VMEM((NCH, CHUNK), jnp.int32),
            pltpu.VMEM((NCH, CHUNK, D), jnp.float32),
            pltpu.VMEM_SHARED((NPAD, D), jnp.float32),
            pltpu.VMEM_SHARED((N, D), jnp.float32),
            pltpu.SemaphoreType.DMA,
            pltpu.SemaphoreType.DMA,
        ],
    )
    def k(table_h, gidx_h, sidx_h, zeros_h, out_h, gv, sv, rows, accum,
          shtab, sem, sem2):
        c = lax.axis_index("c")
        s = lax.axis_index("s")
        wid = s * NC + c
        pltpu.sync_copy(table_h.at[pl.ds(s * 625, 625)],
                        shtab.at[pl.ds(s * 625, 625)])
        pltpu.sync_copy(gidx_h.at[wid], gv)
        pltpu.sync_copy(sidx_h.at[wid], sv)
        pltpu.sync_copy(zeros_h.at[pl.ds(s * RPT, RPT)],
                        accum.at[pl.ds(s * RPT, RPT)])

        plsc.subcore_barrier()

        def fire(j, carry):
            pltpu.async_copy(shtab.at[gv.at[j]], rows.at[j], sem)
            return carry
        lax.fori_loop(0, NCH, fire, 0)

        def drain(j, carry):
            pltpu.make_async_copy(shtab.at[gv.at[j]], rows.at[j], sem).wait()
            return carry
        lax.fori_loop(0, NCH, drain, 0)

        def scat(j, carry):
            pltpu.async_copy(rows.at[j], accum.at[sv.at[j]], sem2, add=True)
            return carry
        lax.fori_loop(0, NCH, scat, 0)

        def sdrain(j, carry):
            pltpu.make_async_copy(rows.at[j], accum.at[sv.at[j]],
                                  sem2).wait()
            return carry
        lax.fori_loop(0, NCH, sdrain, 0)

        plsc.subcore_barrier()
        @pl.when(s < 10)
        def _():
            pltpu.sync_copy(accum.at[pl.ds(s * 1000, 1000)],
                            out_h.at[c].at[pl.ds(s * 1000, 1000)])

    return k(table, gidx, sidx, zeros)


_RA = 5000
_GA = (N // _RA,)
NP128 = N // 16


def _full(shape):
    return pl.BlockSpec(shape, lambda i: tuple(0 for _ in shape))


def _rows(shape):
    if len(shape) == 2:
        return pl.BlockSpec(shape, lambda i: (i, 0))
    return pl.BlockSpec(shape, lambda i: (0, i, 0))


def _tc_a(x, w, Wx, bx, Ww1, bw1, Ww2, bw2):
    def body(x_r, w_r, Wx_r, bx_r, Ww1_r, bw1_r, Ww2_r, bw2_r,
             xa_r, wa_r, wb_r):
        xa_r[...] = jnp.dot(x_r[...], Wx_r[...],
                            preferred_element_type=jnp.float32) + bx_r[...]
        wa_r[...] = jnp.dot(w_r[...], Ww1_r[...],
                            preferred_element_type=jnp.float32) + bw1_r[...]
        wb_r[...] = jnp.dot(w_r[...], Ww2_r[...],
                            preferred_element_type=jnp.float32) + bw2_r[...]

    return pl.pallas_call(
        body,
        grid=_GA,
        in_specs=[_rows((_RA, 256)), _rows((_RA, 128)),
                  _full((256, 8)), _full((1, 8)),
                  _full((128, 8)), _full((1, 8)),
                  _full((128, 8)), _full((1, 8))],
        out_specs=[_rows((_RA, 8))] * 3,
        out_shape=[jax.ShapeDtypeStruct((N, 8), jnp.float32)] * 3,
    )(x, w, Wx, bx, Ww1, bw1, Ww2, bw2)


def _tc_b(xa, parts, W2bd, b2t):
    def body(xa_r, p_r, W2_r, b2_r, x1_r, xc_r):
        x1 = xa_r[...] * (1.0 + p_r[0] + p_r[1])
        x1_r[...] = x1
        xc_r[...] = jnp.dot(x1, W2_r[...],
                            preferred_element_type=jnp.float32) + b2_r[...]

    return pl.pallas_call(
        body,
        grid=(1,),
        in_specs=[_rows((NP128, 128)), _rows((NC, NP128, 128)),
                  _full((128, 128)), _full((1, 128))],
        out_specs=[_rows((NP128, 128))] * 2,
        out_shape=[jax.ShapeDtypeStruct((NP128, 128), jnp.float32)] * 2,
    )(xa, parts, W2bd, b2t)


def _tc_c(x1, wb, parts, Wdbd, bdt, Webd, bet, Wxebd, bxet):
    def body(x1_r, wb_r, p_r, Wd_r, bd_r, We_r, be_r, Wxe_r, bxe_r,
             wd_r, we_r, xe_r):
        w1 = wb_r[...] * (1.0 + p_r[0] + p_r[1])
        w1s = jax.nn.sigmoid(w1)
        wd_r[...] = jnp.dot(w1s, Wd_r[...],
                            preferred_element_type=jnp.float32) + bd_r[...]
        we_r[...] = jnp.dot(w1s, We_r[...],
                            preferred_element_type=jnp.float32) + be_r[...]
        x1s = jax.nn.sigmoid(x1_r[...])
        xe_r[...] = jnp.dot(x1s, Wxe_r[...],
                            preferred_element_type=jnp.float32) + bxe_r[...]

    return pl.pallas_call(
        body,
        grid=(1,),
        in_specs=[_rows((NP128, 128)), _rows((NP128, 128)),
                  _rows((NC, NP128, 128)),
                  _full((128, 128)), _full((1, 128)),
                  _full((128, 128)), _full((1, 128)),
                  _full((128, 128)), _full((1, 128))],
        out_specs=[_rows((NP128, 128))] * 3,
        out_shape=[jax.ShapeDtypeStruct((NP128, 128), jnp.float32)] * 3,
    )(x1, wb, parts, Wdbd, bdt, Webd, bet, Wxebd, bxet)


def _tc_d(xe, parts, Wfbd, bft):
    def body(xe_r, p_r, Wf_r, bf_r, xf_r, x2s_r):
        x2 = xe_r[...] * (1.0 + p_r[0] + p_r[1])
        xf_r[...] = jnp.dot(x2, Wf_r[...],
                            preferred_element_type=jnp.float32) + bf_r[...]
        x2s_r[...] = jax.nn.sigmoid(x2)

    return pl.pallas_call(
        body,
        grid=(1,),
        in_specs=[_rows((NP128, 128)), _rows((NC, NP128, 128)),
                  _full((128, 128)), _full((1, 128))],
        out_specs=[_rows((NP128, 128))] * 2,
        out_shape=[jax.ShapeDtypeStruct((NP128, 128), jnp.float32)] * 2,
    )(xe, parts, Wfbd, bft)


def _tc_e(we, parts):
    def body(we_r, p_r, w2s_r):
        w2s_r[...] = jax.nn.sigmoid(
            we_r[...] * (1.0 + p_r[0] + p_r[1]))

    return pl.pallas_call(
        body,
        grid=(1,),
        in_specs=[_rows((NP128, 128)), _rows((NC, NP128, 128))],
        out_specs=_rows((NP128, 128)),
        out_shape=jax.ShapeDtypeStruct((NP128, 128), jnp.float32),
    )(we, parts)


def kernel(x, h, w,
           c1_mtx_Wx, c1_mtx_bx, c1_mtx_Ww, c1_mtx_bw,
           c1_mte_Wx, c1_mte_bx, c1_mte_Ww, c1_mte_bw,
           c2_mtx_Wx, c2_mtx_bx, c2_mtx_Ww, c2_mtx_bw,
           c2_mte_Wx, c2_mte_bx, c2_mte_Ww, c2_mte_bw):
    h = h.astype(jnp.int32)
    pad = EPAD - E
    g0 = jnp.pad(h[0], (0, pad)).reshape(NW, NCH, CHUNK)
    s0 = jnp.pad(h[0], (0, pad), constant_values=N).reshape(NW, NCH, CHUNK)
    g1 = jnp.pad(h[1], (0, pad)).reshape(NW, NCH, CHUNK)
    s1 = jnp.pad(h[1], (0, pad), constant_values=N).reshape(NW, NCH, CHUNK)
    z8 = jnp.zeros((NPAD, 8), jnp.float32)

    r = lambda b: b.reshape(1, -1)
    p8 = lambda W: jnp.pad(W, [(0, 8 - W.shape[0]), (0, 8 - W.shape[1])])
    bd = lambda W8: jnp.kron(jnp.eye(16, dtype=jnp.float32), W8)
    bt = lambda b8: jnp.tile(b8, 16).reshape(1, 128)
    unpk = lambda a: a.reshape(N, 8)
    pk = lambda p: p.reshape(NC, NP128, 128)

    xa, wa, wb = _tc_a(x, w, c1_mtx_Wx, r(c1_mtx_bx), c1_mtx_Ww, r(c1_mtx_bw),
                       c1_mte_Ww, r(c1_mte_bw))
    xa = xa.reshape(NP128, 128)
    wb = wb.reshape(NP128, 128)
    p1 = _sc_segsum(wa, g1, s0, z8, 8)
    x1, xc = _tc_b(xa, pk(p1), bd(c1_mte_Wx), bt(c1_mte_bx))
    p2 = _sc_segsum(unpk(xc), g0, s1, z8, 8)
    wd, we, xe = _tc_c(x1, wb, pk(p2),
                       bd(p8(c2_mtx_Ww)), bt(p8(c2_mtx_bw[None])[0]),
                       bd(p8(c2_mte_Ww)), bt(p8(c2_mte_bw[None])[0]),
                       bd(p8(c2_mtx_Wx)), bt(p8(c2_mtx_bx[None])[0]))
    p3 = _sc_segsum(unpk(wd), g1, s0, z8, 8)
    xf, x2s = _tc_d(xe, pk(p3), bd(p8(c2_mte_Wx)), bt(p8(c2_mte_bx[None])[0]))
    p4 = _sc_segsum(unpk(xf), g0, s1, z8, 8)
    w2s = _tc_e(we, pk(p4))
    return (unpk(x2s)[:, :4], unpk(w2s)[:, :4])
